# baseline (speedup 1.0000x reference)
import functools

import jax
import jax.numpy as jnp
from jax import lax
from jax.experimental import pallas as pl
from jax.experimental.pallas import tpu as pltpu
from jax.experimental.pallas import tpu_sc as plsc

B = 4096
WLEN = 200
ELEN = 50
DIM = 64
NUM_CLASSES = 16

NC = 2
NS = 16
NW = NC * NS
BPW = B // NW

W_SPLIT = (104, 96)
W_OFFS = tuple(range(0, 192, 16)) + (184,)
E_OFFS = (0, 16, 32, 40)


def _repack_body(src_ref, dst_ref):
  x = src_ref[...]
  dst_ref[:, :DIM] = x
  dst_ref[:, DIM:] = x


def _repack(table, rows_per_block):
  v = table.shape[0]
  return pl.pallas_call(
      _repack_body,
      grid=(v // rows_per_block,),
      in_specs=[pl.BlockSpec((rows_per_block, DIM), lambda i: (i, 0))],
      out_specs=pl.BlockSpec((rows_per_block, 2 * DIM), lambda i: (i, 0)),
      out_shape=jax.ShapeDtypeStruct((v, 2 * DIM), jnp.float32),
  )(table)


G = 8
NG = BPW // G
GW = G * WLEN
GE = G * ELEN
E_CHUNKS = ((0, 128), (128, 128), (256, 128), (384, 16))


def _sc_body(wids_flat, eids_flat, wtab_hbm, etab_hbm,
             wsum_hbm, ent_hbm,
             idx_w, idx_e, rw, ent_v, wsum_v,
             sem_wa, sem_wb, sem_e):
  wid = lax.axis_index("s") * NC + lax.axis_index("c")
  base = wid * BPW
  sems = (sem_wa, sem_wb)

  def fire_words(r, buf):
    c0 = pltpu.async_copy(
        wtab_hbm.at[idx_w.at[pl.ds(r * WLEN, W_SPLIT[0])]],
        rw.at[buf, pl.ds(0, W_SPLIT[0])], sems[buf])
    c1 = pltpu.async_copy(
        wtab_hbm.at[idx_w.at[pl.ds(r * WLEN + W_SPLIT[0], W_SPLIT[1])]],
        rw.at[buf, pl.ds(W_SPLIT[0], W_SPLIT[1])], sems[buf])
    return c0, c1

  def group(g, carry):
    b0 = base + g * G
    pltpu.sync_copy(wids_flat.at[pl.ds(b0 * WLEN, GW)], idx_w)
    pltpu.sync_copy(eids_flat.at[pl.ds(b0 * ELEN, GE)], idx_e)
    ents = [
        pltpu.async_copy(
            etab_hbm.at[idx_e.at[pl.ds(off, n)]],
            ent_v.at[pl.ds(off, n)], sem_e)
        for off, n in E_CHUNKS
    ]
    word_copies = fire_words(0, 0)
    for r in range(G):
      buf = r % 2
      nxt = fire_words(r + 1, 1 - buf) if r + 1 < G else None
      for c in word_copies:
        c.wait()
      word_copies = nxt

      def red(j, acc, _buf=buf):
        return tuple(
            acc[k] + rw[_buf, j, pl.ds(16 * k, 16)] for k in range(4))

      acc = lax.fori_loop(
          0, WLEN, red,
          tuple(jnp.zeros((16,), jnp.float32) for _ in range(4)))
      for k in range(4):
        wsum_v[g * G + r, pl.ds(16 * k, 16)] = acc[k]
    for c in ents:
      c.wait()
    pltpu.sync_copy(ent_v, ent_hbm.at[pl.ds(b0 * ELEN, GE)])
    return carry

  lax.fori_loop(0, NG, group, 0)
  pltpu.sync_copy(wsum_v, wsum_hbm.at[pl.ds(base, BPW)])


@functools.cache
def _sc_gather():
  return pl.kernel(
      _sc_body,
      out_type=(
          jax.ShapeDtypeStruct((B, DIM), jnp.float32),
          jax.ShapeDtypeStruct((B * ELEN, 2 * DIM), jnp.float32),
      ),
      mesh=plsc.VectorSubcoreMesh(core_axis_name="c", subcore_axis_name="s"),
      scratch_types=[
          pltpu.VMEM((GW,), jnp.int32),
          pltpu.VMEM((GE,), jnp.int32),
          pltpu.VMEM((2, WLEN, 2 * DIM), jnp.float32),
          pltpu.VMEM((GE, 2 * DIM), jnp.float32),
          pltpu.VMEM((BPW, DIM), jnp.float32),
          pltpu.SemaphoreType.DMA,
          pltpu.SemaphoreType.DMA,
          pltpu.SemaphoreType.DMA,
      ],
  )


BB = 256


def _tc_body(wids_ref, eids_ref, prior_ref, wsum_ref, ent_ref,
             attw_ref, attb_ref, outw_ref, outb_ref, o_ref):
  wsum = wsum_ref[...]
  eids = eids_ref[...]
  ent = ent_ref[...].reshape(BB, ELEN, 2 * DIM)[:, :, :DIM]
  nonzero = jnp.sum((wids_ref[...] != 0).astype(jnp.float32), axis=1,
                    keepdims=True)
  w_norm = jnp.maximum(
      jnp.sqrt(jnp.sum(wsum * wsum, axis=1, keepdims=True)), 1e-12)
  wn = wsum / w_norm
  e_norm = jnp.maximum(
      jnp.sqrt(jnp.sum(ent * ent, axis=2)), 1e-12)
  cos = jnp.sum(wn[:, None, :] * ent, axis=2) / e_norm
  logits = (prior_ref[...] * attw_ref[0, 0] + cos * attw_ref[0, 1]
            + attb_ref[0])
  logits = jnp.where(eids == 0, -1e32, logits)
  m = jnp.max(logits, axis=1, keepdims=True)
  e = jnp.exp(logits - m)
  aw = e / jnp.sum(e, axis=1, keepdims=True)
  feat = jnp.sum(ent * aw[:, :, None], axis=1)
  feat = feat + wsum / nonzero
  o_ref[...] = lax.dot_general(
      feat, outw_ref[...], (((1,), (1,)), ((), ())),
      preferred_element_type=jnp.float32) + outb_ref[...]


def _tc_dense(word_ids, entity_ids, prior_probs, wsum, ent,
              att_w, att_b, out_w, out_b):
  grid = B // BB
  return pl.pallas_call(
      _tc_body,
      grid=(grid,),
      in_specs=[
          pl.BlockSpec((BB, WLEN), lambda i: (i, 0)),
          pl.BlockSpec((BB, ELEN), lambda i: (i, 0)),
          pl.BlockSpec((BB, ELEN), lambda i: (i, 0)),
          pl.BlockSpec((BB, DIM), lambda i: (i, 0)),
          pl.BlockSpec((BB * ELEN, 2 * DIM), lambda i: (i, 0)),
          pl.BlockSpec(memory_space=pltpu.SMEM),
          pl.BlockSpec(memory_space=pltpu.SMEM),
          pl.BlockSpec((NUM_CLASSES, DIM), lambda i: (0, 0)),
          pl.BlockSpec((1, NUM_CLASSES), lambda i: (0, 0)),
      ],
      out_specs=pl.BlockSpec((BB, NUM_CLASSES), lambda i: (i, 0)),
      out_shape=jax.ShapeDtypeStruct((B, NUM_CLASSES), jnp.float32),
  )(word_ids, entity_ids, prior_probs, wsum, ent,
    att_w, att_b, out_w, out_b)


def kernel(word_ids, entity_ids, prior_probs, word_table, entity_table,
           att_w, att_b, out_w, out_b):
  wpack = _repack(word_table, 8000)
  epack = _repack(entity_table, 4000)
  wsum, ent = _sc_gather()(word_ids.reshape(B * WLEN),
                           entity_ids.reshape(B * ELEN), wpack, epack)
  return _tc_dense(word_ids, entity_ids, prior_probs, wsum, ent,
                   att_w, att_b, out_w, out_b.reshape(1, NUM_CLASSES))


# speedup vs baseline: 1.1690x; 1.1690x over previous
import functools

import jax
import jax.numpy as jnp
from jax import lax
from jax.experimental import pallas as pl
from jax.experimental.pallas import tpu as pltpu
from jax.experimental.pallas import tpu_sc as plsc

B = 4096
WLEN = 200
ELEN = 50
DIM = 64
NUM_CLASSES = 16

NC = 2
NS = 16
NW = NC * NS
BPW = B // NW

W_SPLIT = (104, 96)
W_OFFS = tuple(range(0, 192, 16)) + (184,)
E_OFFS = (0, 16, 32, 40)


def _widen(table):
  return jnp.pad(table, ((0, 0), (0, DIM)))


G = 8
NG = BPW // G
GW = G * WLEN
GE = G * ELEN
E_CHUNKS = ((0, 128), (128, 128), (256, 128), (384, 16))


def _sc_body(wids_flat, eids_flat, wtab_hbm, etab_hbm,
             wsum_hbm, ent_hbm,
             idx_w, idx_e, rw, ent_v, wsum_v,
             sem_wa, sem_wb, sem_e):
  wid = lax.axis_index("s") * NC + lax.axis_index("c")
  base = wid * BPW
  sems = (sem_wa, sem_wb)

  def fire_words(r, buf):
    c0 = pltpu.async_copy(
        wtab_hbm.at[idx_w.at[pl.ds(r * WLEN, W_SPLIT[0])]],
        rw.at[buf, pl.ds(0, W_SPLIT[0])], sems[buf])
    c1 = pltpu.async_copy(
        wtab_hbm.at[idx_w.at[pl.ds(r * WLEN + W_SPLIT[0], W_SPLIT[1])]],
        rw.at[buf, pl.ds(W_SPLIT[0], W_SPLIT[1])], sems[buf])
    return c0, c1

  def group(g, carry):
    b0 = base + g * G
    pltpu.sync_copy(wids_flat.at[pl.ds(b0 * WLEN, GW)], idx_w)
    pltpu.sync_copy(eids_flat.at[pl.ds(b0 * ELEN, GE)], idx_e)
    ents = [
        pltpu.async_copy(
            etab_hbm.at[idx_e.at[pl.ds(off, n)]],
            ent_v.at[pl.ds(off, n)], sem_e)
        for off, n in E_CHUNKS
    ]
    word_copies = fire_words(0, 0)
    for r in range(G):
      buf = r % 2
      nxt = fire_words(r + 1, 1 - buf) if r + 1 < G else None
      for c in word_copies:
        c.wait()
      word_copies = nxt

      def red(j, acc, _buf=buf):
        return tuple(
            acc[k] + rw[_buf, j, pl.ds(16 * k, 16)] for k in range(4))

      acc = lax.fori_loop(
          0, WLEN, red,
          tuple(jnp.zeros((16,), jnp.float32) for _ in range(4)))
      for k in range(4):
        wsum_v[g * G + r, pl.ds(16 * k, 16)] = acc[k]
    for c in ents:
      c.wait()
    pltpu.sync_copy(ent_v, ent_hbm.at[pl.ds(b0 * ELEN, GE)])
    return carry

  lax.fori_loop(0, NG, group, 0)
  pltpu.sync_copy(wsum_v, wsum_hbm.at[pl.ds(base, BPW)])


@functools.cache
def _sc_gather():
  return pl.kernel(
      _sc_body,
      out_type=(
          jax.ShapeDtypeStruct((B, DIM), jnp.float32),
          jax.ShapeDtypeStruct((B * ELEN, 2 * DIM), jnp.float32),
      ),
      mesh=plsc.VectorSubcoreMesh(core_axis_name="c", subcore_axis_name="s"),
      scratch_types=[
          pltpu.VMEM((GW,), jnp.int32),
          pltpu.VMEM((GE,), jnp.int32),
          pltpu.VMEM((2, WLEN, 2 * DIM), jnp.float32),
          pltpu.VMEM((GE, 2 * DIM), jnp.float32),
          pltpu.VMEM((BPW, DIM), jnp.float32),
          pltpu.SemaphoreType.DMA,
          pltpu.SemaphoreType.DMA,
          pltpu.SemaphoreType.DMA,
      ],
  )


BB = 256


def _tc_body(wids_ref, eids_ref, prior_ref, wsum_ref, ent_ref,
             attw_ref, attb_ref, outw_ref, outb_ref, o_ref):
  wsum = wsum_ref[...]
  eids = eids_ref[...]
  ent = ent_ref[...].reshape(BB, ELEN, 2 * DIM)[:, :, :DIM]
  nonzero = jnp.sum((wids_ref[...] != 0).astype(jnp.float32), axis=1,
                    keepdims=True)
  w_norm = jnp.maximum(
      jnp.sqrt(jnp.sum(wsum * wsum, axis=1, keepdims=True)), 1e-12)
  wn = wsum / w_norm
  e_norm = jnp.maximum(
      jnp.sqrt(jnp.sum(ent * ent, axis=2)), 1e-12)
  cos = jnp.sum(wn[:, None, :] * ent, axis=2) / e_norm
  logits = (prior_ref[...] * attw_ref[0, 0] + cos * attw_ref[0, 1]
            + attb_ref[0])
  logits = jnp.where(eids == 0, -1e32, logits)
  m = jnp.max(logits, axis=1, keepdims=True)
  e = jnp.exp(logits - m)
  aw = e / jnp.sum(e, axis=1, keepdims=True)
  feat = jnp.sum(ent * aw[:, :, None], axis=1)
  feat = feat + wsum / nonzero
  o_ref[...] = lax.dot_general(
      feat, outw_ref[...], (((1,), (1,)), ((), ())),
      preferred_element_type=jnp.float32) + outb_ref[...]


def _tc_dense(word_ids, entity_ids, prior_probs, wsum, ent,
              att_w, att_b, out_w, out_b):
  grid = B // BB
  return pl.pallas_call(
      _tc_body,
      grid=(grid,),
      in_specs=[
          pl.BlockSpec((BB, WLEN), lambda i: (i, 0)),
          pl.BlockSpec((BB, ELEN), lambda i: (i, 0)),
          pl.BlockSpec((BB, ELEN), lambda i: (i, 0)),
          pl.BlockSpec((BB, DIM), lambda i: (i, 0)),
          pl.BlockSpec((BB * ELEN, 2 * DIM), lambda i: (i, 0)),
          pl.BlockSpec(memory_space=pltpu.SMEM),
          pl.BlockSpec(memory_space=pltpu.SMEM),
          pl.BlockSpec((NUM_CLASSES, DIM), lambda i: (0, 0)),
          pl.BlockSpec((1, NUM_CLASSES), lambda i: (0, 0)),
      ],
      out_specs=pl.BlockSpec((BB, NUM_CLASSES), lambda i: (i, 0)),
      out_shape=jax.ShapeDtypeStruct((B, NUM_CLASSES), jnp.float32),
  )(word_ids, entity_ids, prior_probs, wsum, ent,
    att_w, att_b, out_w, out_b)


def kernel(word_ids, entity_ids, prior_probs, word_table, entity_table,
           att_w, att_b, out_w, out_b):
  wpack = _widen(word_table)
  epack = _widen(entity_table)
  wsum, ent = _sc_gather()(word_ids.reshape(B * WLEN),
                           entity_ids.reshape(B * ELEN), wpack, epack)
  return _tc_dense(word_ids, entity_ids, prior_probs, wsum, ent,
                   att_w, att_b, out_w, out_b.reshape(1, NUM_CLASSES))
